# Initial kernel scaffold; baseline (speedup 1.0000x reference)
import jax
import jax.numpy as jnp
from jax.experimental import pallas as pl


def kernel(ufeat, ifeat, edge_index):
    raise NotImplementedError("write your pallas kernel here")


# speedup vs baseline: 1.0609x; 1.0609x over previous
import dataclasses
import functools

import jax
import jax.numpy as jnp
from jax.experimental import pallas as pl
from jax.experimental.pallas import tpu as pltpu
from jax.experimental.pallas import tpu_sc as plsc

E = 320000
D = 128
LANES = 16
CHUNK = 128
NCHUNK = E // CHUNK


def _sc_dot_kernel(u_hbm, i_hbm, s_hbm, d_hbm, o_hbm, u_rows, i_rows, sem_u, sem_i):
    def body(s_vmem, d_vmem, o_vmem):
        cu = pltpu.async_copy(u_hbm.at[s_vmem.at[0]], u_rows, sem_u)
        ci = pltpu.async_copy(i_hbm.at[d_vmem.at[0]], i_rows, sem_i)
        cu.wait()
        ci.wait()

        @pl.loop(0, CHUNK, step=LANES)
        def _(e0):
            rows = e0 + jax.lax.iota(jnp.int32, LANES)
            accs = [jnp.zeros((LANES,), jnp.float32) for _ in range(4)]
            for d in range(D):
                col = jnp.full((LANES,), d, jnp.int32)
                uv = plsc.load_gather(u_rows, [rows, col])
                iv = plsc.load_gather(i_rows, [rows, col])
                accs[d % 4] = accs[d % 4] + uv * iv
            o_vmem[0, pl.ds(e0, LANES)] = (accs[0] + accs[1]) + (accs[2] + accs[3])

    pltpu.emit_pipeline(
        body,
        grid=(NCHUNK,),
        in_specs=[
            pl.BlockSpec((1, CHUNK), lambda i: (0, i)),
            pl.BlockSpec((1, CHUNK), lambda i: (0, i)),
        ],
        out_specs=[pl.BlockSpec((1, CHUNK), lambda i: (i, 0))],
        core_axis_name=("core", "subcore"),
        dimension_semantics=(pltpu.PARALLEL,),
    )(s_hbm, d_hbm, o_hbm)


def kernel(ufeat, ifeat, edge_index):
    src = edge_index[0].astype(jnp.int32).reshape(1, E)
    dst = edge_index[1].astype(jnp.int32).reshape(1, E)
    mesh = plsc.VectorSubcoreMesh(core_axis_name="core", subcore_axis_name="subcore")

    cp = pltpu.CompilerParams()
    if "needs_layout_passes" in pltpu.CompilerParams.__dataclass_fields__:
        cp = dataclasses.replace(cp, needs_layout_passes=False)

    run = functools.partial(
        pl.kernel,
        out_type=jax.ShapeDtypeStruct((NCHUNK, CHUNK), jnp.float32),
        mesh=mesh,
        compiler_params=cp,
        scratch_types=[
            pltpu.VMEM((CHUNK, D), jnp.float32),
            pltpu.VMEM((CHUNK, D), jnp.float32),
            pltpu.SemaphoreType.DMA,
            pltpu.SemaphoreType.DMA,
        ],
    )(_sc_dot_kernel)

    out = run(ufeat, ifeat, src, dst)
    return out.reshape(E, 1)


# speedup vs baseline: 1.1710x; 1.1038x over previous
import dataclasses
import functools

import jax
import jax.numpy as jnp
from jax import lax
from jax.experimental import pallas as pl
from jax.experimental.pallas import tpu as pltpu
from jax.experimental.pallas import tpu_sc as plsc

E = 320000
N = 10000
D = 128
LANES = 16
NWORKER = 32
SPAN = E // NWORKER
CHUNK = 80
NCHUNK = SPAN // CHUNK


def _dot_chunk(u_rows, i_rows, out_v, t):
    @pl.loop(0, CHUNK, step=LANES)
    def _(e0):
        rows = e0 + lax.iota(jnp.int32, LANES)
        accs = [jnp.zeros((LANES,), jnp.float32) for _ in range(4)]
        for d in range(D):
            col = jnp.full((LANES,), d, jnp.int32)
            uv = plsc.load_gather(u_rows, [rows, col])
            iv = plsc.load_gather(i_rows, [rows, col])
            accs[d % 4] = accs[d % 4] + uv * iv
        out_v[pl.ds(t * CHUNK + e0, LANES)] = (accs[0] + accs[1]) + (accs[2] + accs[3])


def _sc_dot_kernel(
    u_hbm, i_hbm, s_hbm, d_hbm, o_hbm,
    sidx, didx, u0, i0, u1, i1, out_v,
    sem_idx, su0, si0, su1, si1,
):
    w = lax.axis_index("core") * 16 + lax.axis_index("subcore")
    base = w * SPAN

    pltpu.async_copy(s_hbm.at[pl.ds(base, SPAN)], sidx, sem_idx).wait()
    pltpu.async_copy(d_hbm.at[pl.ds(base, SPAN)], didx, sem_idx).wait()

    def fire(t, u_buf, i_buf, sem_u, sem_i):
        pltpu.async_copy(u_hbm.at[sidx.at[pl.ds(t * CHUNK, CHUNK)]], u_buf, sem_u)
        pltpu.async_copy(i_hbm.at[didx.at[pl.ds(t * CHUNK, CHUNK)]], i_buf, sem_i)

    def drain(u_buf, i_buf, sem_u, sem_i):
        pltpu.make_async_copy(u_hbm.at[sidx.at[pl.ds(0, CHUNK)]], u_buf, sem_u).wait()
        pltpu.make_async_copy(i_hbm.at[didx.at[pl.ds(0, CHUNK)]], i_buf, sem_i).wait()

    fire(0, u0, i0, su0, si0)

    @pl.loop(0, NCHUNK - 1, step=2)
    def _(t):
        drain(u0, i0, su0, si0)
        fire(t + 1, u1, i1, su1, si1)
        _dot_chunk(u0, i0, out_v, t)
        drain(u1, i1, su1, si1)
        fire(t + 2, u0, i0, su0, si0)
        _dot_chunk(u1, i1, out_v, t + 1)

    drain(u0, i0, su0, si0)
    _dot_chunk(u0, i0, out_v, NCHUNK - 1)

    pltpu.sync_copy(out_v, o_hbm.at[pl.ds(base, SPAN)])


def kernel(ufeat, ifeat, edge_index):
    src = edge_index[0].astype(jnp.int32)
    dst = edge_index[1].astype(jnp.int32)
    mesh = plsc.VectorSubcoreMesh(core_axis_name="core", subcore_axis_name="subcore")

    cp = pltpu.CompilerParams()
    if "needs_layout_passes" in pltpu.CompilerParams.__dataclass_fields__:
        cp = dataclasses.replace(cp, needs_layout_passes=False)

    run = functools.partial(
        pl.kernel,
        out_type=jax.ShapeDtypeStruct((E,), jnp.float32),
        mesh=mesh,
        compiler_params=cp,
        scratch_types=[
            pltpu.VMEM((SPAN,), jnp.int32),
            pltpu.VMEM((SPAN,), jnp.int32),
            pltpu.VMEM((CHUNK, D), jnp.float32),
            pltpu.VMEM((CHUNK, D), jnp.float32),
            pltpu.VMEM((CHUNK, D), jnp.float32),
            pltpu.VMEM((CHUNK, D), jnp.float32),
            pltpu.VMEM((SPAN,), jnp.float32),
            pltpu.SemaphoreType.DMA,
            pltpu.SemaphoreType.DMA,
            pltpu.SemaphoreType.DMA,
            pltpu.SemaphoreType.DMA,
            pltpu.SemaphoreType.DMA,
        ],
    )(_sc_dot_kernel)

    out = run(ufeat, ifeat, src, dst)
    return out.reshape(E, 1)


# speedup vs baseline: 2.9762x; 2.5416x over previous
import dataclasses
import functools

import jax
import jax.numpy as jnp
from jax import lax
from jax.experimental import pallas as pl
from jax.experimental.pallas import tpu as pltpu
from jax.experimental.pallas import tpu_sc as plsc

E = 320000
N = 10000
D = 128
LANES = 16
NWORKER = 32
SPAN = E // NWORKER
CHUNK = 80
NCHUNK = SPAN // CHUNK


def _dot_chunk(u_rows, i_rows, out_v, t):
    @pl.loop(0, CHUNK, step=LANES)
    def _(e0):
        lane = lax.iota(jnp.int32, LANES)
        rows = e0 + lane
        accs = [jnp.zeros((LANES,), jnp.float32) for _ in range(4)]
        for d in range(D):
            col = (lane + d) & (D - 1)
            uv = plsc.load_gather(u_rows, [rows, col])
            iv = plsc.load_gather(i_rows, [rows, col])
            accs[d % 4] = accs[d % 4] + uv * iv
        out_v[pl.ds(t * CHUNK + e0, LANES)] = (accs[0] + accs[1]) + (accs[2] + accs[3])


def _sc_dot_kernel(
    u_hbm, i_hbm, s_hbm, d_hbm, o_hbm,
    sidx, didx, u0, i0, u1, i1, out_v,
    sem_idx, su0, si0, su1, si1,
):
    w = lax.axis_index("core") * 16 + lax.axis_index("subcore")
    base = w * SPAN

    pltpu.async_copy(s_hbm.at[pl.ds(base, SPAN)], sidx, sem_idx).wait()
    pltpu.async_copy(d_hbm.at[pl.ds(base, SPAN)], didx, sem_idx).wait()

    def fire(t, u_buf, i_buf, sem_u, sem_i):
        pltpu.async_copy(u_hbm.at[sidx.at[pl.ds(t * CHUNK, CHUNK)]], u_buf, sem_u)
        pltpu.async_copy(i_hbm.at[didx.at[pl.ds(t * CHUNK, CHUNK)]], i_buf, sem_i)

    def drain(u_buf, i_buf, sem_u, sem_i):
        pltpu.make_async_copy(u_hbm.at[sidx.at[pl.ds(0, CHUNK)]], u_buf, sem_u).wait()
        pltpu.make_async_copy(i_hbm.at[didx.at[pl.ds(0, CHUNK)]], i_buf, sem_i).wait()

    fire(0, u0, i0, su0, si0)

    @pl.loop(0, NCHUNK - 1, step=2)
    def _(t):
        drain(u0, i0, su0, si0)
        fire(t + 1, u1, i1, su1, si1)
        _dot_chunk(u0, i0, out_v, t)
        drain(u1, i1, su1, si1)
        fire(t + 2, u0, i0, su0, si0)
        _dot_chunk(u1, i1, out_v, t + 1)

    drain(u0, i0, su0, si0)
    _dot_chunk(u0, i0, out_v, NCHUNK - 1)

    pltpu.sync_copy(out_v, o_hbm.at[pl.ds(base, SPAN)])


def kernel(ufeat, ifeat, edge_index):
    src = edge_index[0].astype(jnp.int32)
    dst = edge_index[1].astype(jnp.int32)
    mesh = plsc.VectorSubcoreMesh(core_axis_name="core", subcore_axis_name="subcore")

    cp = pltpu.CompilerParams()
    if "needs_layout_passes" in pltpu.CompilerParams.__dataclass_fields__:
        cp = dataclasses.replace(cp, needs_layout_passes=False)

    run = functools.partial(
        pl.kernel,
        out_type=jax.ShapeDtypeStruct((E,), jnp.float32),
        mesh=mesh,
        compiler_params=cp,
        scratch_types=[
            pltpu.VMEM((SPAN,), jnp.int32),
            pltpu.VMEM((SPAN,), jnp.int32),
            pltpu.VMEM((CHUNK, D), jnp.float32),
            pltpu.VMEM((CHUNK, D), jnp.float32),
            pltpu.VMEM((CHUNK, D), jnp.float32),
            pltpu.VMEM((CHUNK, D), jnp.float32),
            pltpu.VMEM((SPAN,), jnp.float32),
            pltpu.SemaphoreType.DMA,
            pltpu.SemaphoreType.DMA,
            pltpu.SemaphoreType.DMA,
            pltpu.SemaphoreType.DMA,
            pltpu.SemaphoreType.DMA,
        ],
    )(_sc_dot_kernel)

    out = run(ufeat, ifeat, src, dst)
    return out.reshape(E, 1)


# speedup vs baseline: 7.7575x; 2.6065x over previous
import dataclasses
import functools

import jax
import jax.numpy as jnp
from jax import lax
from jax.experimental import pallas as pl
from jax.experimental.pallas import tpu as pltpu
from jax.experimental.pallas import tpu_sc as plsc

E = 320000
N = 10000
D = 128
LANES = 16
NWORKER = 32
SPAN = E // NWORKER
CHUNK = 80
NCHUNK = SPAN // CHUNK


def _dot_chunk(u_rows, i_rows, out_v, t):
    UNROLL = 8

    @pl.loop(0, CHUNK, step=LANES)
    def _(e0):
        lane = lax.iota(jnp.int32, LANES)
        rows = e0 + lane

        def dstep(k, accs):
            accs = list(accs)
            d0 = k * UNROLL
            for j in range(UNROLL):
                col = (lane + (d0 + j)) & (D - 1)
                uv = plsc.load_gather(u_rows, [rows, col])
                iv = plsc.load_gather(i_rows, [rows, col])
                accs[j % 4] = accs[j % 4] + uv * iv
            return tuple(accs)

        zero = jnp.zeros((LANES,), jnp.float32)
        accs = lax.fori_loop(0, D // UNROLL, dstep, (zero, zero, zero, zero))
        out_v[pl.ds(t * CHUNK + e0, LANES)] = (accs[0] + accs[1]) + (accs[2] + accs[3])


def _sc_dot_kernel(
    u_hbm, i_hbm, s_hbm, d_hbm, o_hbm,
    sidx, didx, u0, i0, u1, i1, out_v,
    sem_idx, su0, si0, su1, si1,
):
    w = lax.axis_index("core") * 16 + lax.axis_index("subcore")
    base = w * SPAN

    pltpu.async_copy(s_hbm.at[pl.ds(base, SPAN)], sidx, sem_idx).wait()
    pltpu.async_copy(d_hbm.at[pl.ds(base, SPAN)], didx, sem_idx).wait()

    def fire(t, u_buf, i_buf, sem_u, sem_i):
        pltpu.async_copy(u_hbm.at[sidx.at[pl.ds(t * CHUNK, CHUNK)]], u_buf, sem_u)
        pltpu.async_copy(i_hbm.at[didx.at[pl.ds(t * CHUNK, CHUNK)]], i_buf, sem_i)

    def drain(u_buf, i_buf, sem_u, sem_i):
        pltpu.make_async_copy(u_hbm.at[sidx.at[pl.ds(0, CHUNK)]], u_buf, sem_u).wait()
        pltpu.make_async_copy(i_hbm.at[didx.at[pl.ds(0, CHUNK)]], i_buf, sem_i).wait()

    fire(0, u0, i0, su0, si0)

    @pl.loop(0, NCHUNK - 1, step=2)
    def _(t):
        drain(u0, i0, su0, si0)
        fire(t + 1, u1, i1, su1, si1)
        _dot_chunk(u0, i0, out_v, t)
        drain(u1, i1, su1, si1)
        fire(t + 2, u0, i0, su0, si0)
        _dot_chunk(u1, i1, out_v, t + 1)

    drain(u0, i0, su0, si0)
    _dot_chunk(u0, i0, out_v, NCHUNK - 1)

    pltpu.sync_copy(out_v, o_hbm.at[pl.ds(base, SPAN)])


def kernel(ufeat, ifeat, edge_index):
    src = edge_index[0].astype(jnp.int32)
    dst = edge_index[1].astype(jnp.int32)
    mesh = plsc.VectorSubcoreMesh(core_axis_name="core", subcore_axis_name="subcore")

    cp = pltpu.CompilerParams()
    if "needs_layout_passes" in pltpu.CompilerParams.__dataclass_fields__:
        cp = dataclasses.replace(cp, needs_layout_passes=False)

    run = functools.partial(
        pl.kernel,
        out_type=jax.ShapeDtypeStruct((E,), jnp.float32),
        mesh=mesh,
        compiler_params=cp,
        scratch_types=[
            pltpu.VMEM((SPAN,), jnp.int32),
            pltpu.VMEM((SPAN,), jnp.int32),
            pltpu.VMEM((CHUNK, D), jnp.float32),
            pltpu.VMEM((CHUNK, D), jnp.float32),
            pltpu.VMEM((CHUNK, D), jnp.float32),
            pltpu.VMEM((CHUNK, D), jnp.float32),
            pltpu.VMEM((SPAN,), jnp.float32),
            pltpu.SemaphoreType.DMA,
            pltpu.SemaphoreType.DMA,
            pltpu.SemaphoreType.DMA,
            pltpu.SemaphoreType.DMA,
            pltpu.SemaphoreType.DMA,
        ],
    )(_sc_dot_kernel)

    out = run(ufeat, ifeat, src, dst)
    return out.reshape(E, 1)


# speedup vs baseline: 7.8011x; 1.0056x over previous
import dataclasses
import functools

import jax
import jax.numpy as jnp
from jax import lax
from jax.experimental import pallas as pl
from jax.experimental.pallas import tpu as pltpu
from jax.experimental.pallas import tpu_sc as plsc

E = 320000
N = 10000
D = 128
LANES = 16
NWORKER = 32
SPAN = E // NWORKER
CHUNK = 80
NCHUNK = SPAN // CHUNK


def _dot_chunk(u_rows, i_rows, out_v, t):
    UNROLL = 16

    @pl.loop(0, CHUNK, step=LANES)
    def _(e0):
        lane = lax.iota(jnp.int32, LANES)
        rows = e0 + lane

        def dstep(k, accs):
            accs = list(accs)
            d0 = k * UNROLL
            for j in range(UNROLL):
                col = (lane + (d0 + j)) & (D - 1)
                uv = plsc.load_gather(u_rows, [rows, col])
                iv = plsc.load_gather(i_rows, [rows, col])
                accs[j % 4] = accs[j % 4] + uv * iv
            return tuple(accs)

        zero = jnp.zeros((LANES,), jnp.float32)
        accs = lax.fori_loop(0, D // UNROLL, dstep, (zero, zero, zero, zero))
        out_v[pl.ds(t * CHUNK + e0, LANES)] = (accs[0] + accs[1]) + (accs[2] + accs[3])


def _sc_dot_kernel(
    u_hbm, i_hbm, s_hbm, d_hbm, o_hbm,
    sidx, didx, u0, i0, u1, i1, out_v,
    sem_idx, su0, si0, su1, si1,
):
    w = lax.axis_index("core") * 16 + lax.axis_index("subcore")
    base = w * SPAN

    pltpu.async_copy(s_hbm.at[pl.ds(base, SPAN)], sidx, sem_idx).wait()
    pltpu.async_copy(d_hbm.at[pl.ds(base, SPAN)], didx, sem_idx).wait()

    def fire(t, u_buf, i_buf, sem_u, sem_i):
        pltpu.async_copy(u_hbm.at[sidx.at[pl.ds(t * CHUNK, CHUNK)]], u_buf, sem_u)
        pltpu.async_copy(i_hbm.at[didx.at[pl.ds(t * CHUNK, CHUNK)]], i_buf, sem_i)

    def drain(u_buf, i_buf, sem_u, sem_i):
        pltpu.make_async_copy(u_hbm.at[sidx.at[pl.ds(0, CHUNK)]], u_buf, sem_u).wait()
        pltpu.make_async_copy(i_hbm.at[didx.at[pl.ds(0, CHUNK)]], i_buf, sem_i).wait()

    fire(0, u0, i0, su0, si0)

    @pl.loop(0, NCHUNK - 1, step=2)
    def _(t):
        drain(u0, i0, su0, si0)
        fire(t + 1, u1, i1, su1, si1)
        _dot_chunk(u0, i0, out_v, t)
        drain(u1, i1, su1, si1)
        fire(t + 2, u0, i0, su0, si0)
        _dot_chunk(u1, i1, out_v, t + 1)

    drain(u0, i0, su0, si0)
    _dot_chunk(u0, i0, out_v, NCHUNK - 1)

    pltpu.sync_copy(out_v, o_hbm.at[pl.ds(base, SPAN)])


def kernel(ufeat, ifeat, edge_index):
    src = edge_index[0].astype(jnp.int32)
    dst = edge_index[1].astype(jnp.int32)
    mesh = plsc.VectorSubcoreMesh(core_axis_name="core", subcore_axis_name="subcore")

    cp = pltpu.CompilerParams()
    if "needs_layout_passes" in pltpu.CompilerParams.__dataclass_fields__:
        cp = dataclasses.replace(cp, needs_layout_passes=False)

    run = functools.partial(
        pl.kernel,
        out_type=jax.ShapeDtypeStruct((E,), jnp.float32),
        mesh=mesh,
        compiler_params=cp,
        scratch_types=[
            pltpu.VMEM((SPAN,), jnp.int32),
            pltpu.VMEM((SPAN,), jnp.int32),
            pltpu.VMEM((CHUNK, D), jnp.float32),
            pltpu.VMEM((CHUNK, D), jnp.float32),
            pltpu.VMEM((CHUNK, D), jnp.float32),
            pltpu.VMEM((CHUNK, D), jnp.float32),
            pltpu.VMEM((SPAN,), jnp.float32),
            pltpu.SemaphoreType.DMA,
            pltpu.SemaphoreType.DMA,
            pltpu.SemaphoreType.DMA,
            pltpu.SemaphoreType.DMA,
            pltpu.SemaphoreType.DMA,
        ],
    )(_sc_dot_kernel)

    out = run(ufeat, ifeat, src, dst)
    return out.reshape(E, 1)
